# baseline (speedup 1.0000x reference)
import functools

import jax
import jax.numpy as jnp
from jax import lax
from jax.experimental import pallas as pl
from jax.experimental.pallas import tpu as pltpu
from jax.experimental.pallas import tpu_sc as plsc

_NUM_CORES = 2
_NUM_SUBCORES = 16
_LANES = 16
_CHUNK = 2048


@functools.lru_cache(maxsize=None)
def _build_gather(batch, dim, rows, dtype_name):
    dtype = jnp.dtype(dtype_name)
    n_workers = _NUM_CORES * _NUM_SUBCORES
    cols_per_w = dim // n_workers
    n_chunks = batch // _CHUNK
    mesh = plsc.VectorSubcoreMesh(
        core_axis_name="c",
        subcore_axis_name="s",
        num_cores=_NUM_CORES,
        num_subcores=_NUM_SUBCORES,
    )

    @functools.partial(
        pl.kernel,
        mesh=mesh,
        out_type=jax.ShapeDtypeStruct((dim, batch), dtype),
        scratch_types=[
            pltpu.VMEM((rows,), dtype),
            pltpu.VMEM((batch,), jnp.int32),
            pltpu.VMEM((2, _CHUNK), dtype),
            pltpu.SemaphoreType.DMA,
            pltpu.SemaphoreType.DMA,
            pltpu.SemaphoreType.DMA,
        ],
        compiler_params=pltpu.CompilerParams(
            use_tc_tiling_on_sc=True, needs_layout_passes=False
        ),
    )
    def gather_kernel(
        tableT_hbm, idx_hbm, outT_hbm, row_v, idx_v, outc_v, sem0, sem1, semr
    ):
        wid = lax.axis_index("s") * _NUM_CORES + lax.axis_index("c")
        sems = (sem0, sem1)
        idx_copy = pltpu.async_copy(idx_hbm, idx_v, sem0)
        row_copy = pltpu.async_copy(tableT_hbm.at[wid * cols_per_w], row_v, semr)
        idx_copy.wait()

        for r in range(cols_per_w):
            col = wid * cols_per_w + r
            if r > 0:
                row_copy = pltpu.async_copy(tableT_hbm.at[col], row_v, semr)
            row_copy.wait()
            pending = [None, None]
            for k in range(n_chunks):
                b = k % 2
                if pending[b] is not None:
                    pending[b].wait()

                @plsc.parallel_loop(0, _CHUNK // _LANES, unroll=16)
                def per_vec(j, k=k, b=b):
                    sl_out = pl.ds(j * _LANES, _LANES)
                    sl_idx = pl.ds(k * _CHUNK + j * _LANES, _LANES)
                    outc_v[b, sl_out] = plsc.load_gather(row_v, [idx_v[sl_idx]])
                pending[b] = pltpu.async_copy(
                    outc_v.at[b],
                    outT_hbm.at[col, pl.ds(k * _CHUNK, _CHUNK)],
                    sems[b],
                )
            for p in pending:
                if p is not None:
                    p.wait()

    return gather_kernel


@jax.jit
def kernel(x, pe):
    rows, dim = pe.shape
    gather = _build_gather(x.shape[0], dim, rows, pe.dtype.name)
    outT = gather(pe.T, x)
    return outT.T


# speedup vs baseline: 1.0312x; 1.0312x over previous
import functools

import jax
import jax.numpy as jnp
from jax import lax
from jax.experimental import pallas as pl
from jax.experimental.pallas import tpu as pltpu
from jax.experimental.pallas import tpu_sc as plsc

_NUM_CORES = 2
_NUM_SUBCORES = 16
_LANES = 16
_CHUNK = 2048


@functools.lru_cache(maxsize=None)
def _build_gather(batch, dim, rows, dtype_name):
    dtype = jnp.dtype(dtype_name)
    n_workers = _NUM_CORES * _NUM_SUBCORES
    cols_per_w = dim // n_workers
    n_chunks = batch // _CHUNK
    mesh = plsc.VectorSubcoreMesh(
        core_axis_name="c",
        subcore_axis_name="s",
        num_cores=_NUM_CORES,
        num_subcores=_NUM_SUBCORES,
    )

    @functools.partial(
        pl.kernel,
        mesh=mesh,
        out_type=jax.ShapeDtypeStruct((dim, batch), dtype),
        scratch_types=[
            pltpu.VMEM((rows,), dtype),
            pltpu.VMEM((batch,), jnp.int32),
            pltpu.VMEM((2, _CHUNK), dtype),
            pltpu.SemaphoreType.DMA,
            pltpu.SemaphoreType.DMA,
            pltpu.SemaphoreType.DMA,
        ],
        compiler_params=pltpu.CompilerParams(
            use_tc_tiling_on_sc=True, needs_layout_passes=False
        ),
    )
    def gather_kernel(
        tableT_hbm, idx_hbm, outT_hbm, row_v, idx_v, outc_v, sem0, sem1, semr
    ):
        wid = lax.axis_index("s") * _NUM_CORES + lax.axis_index("c")
        sems = (sem0, sem1)
        idx_copy = pltpu.async_copy(idx_hbm, idx_v, sem0)
        row_copy = pltpu.async_copy(tableT_hbm.at[wid * cols_per_w], row_v, semr)
        idx_copy.wait()

        for r in range(cols_per_w):
            col = wid * cols_per_w + r
            if r > 0:
                row_copy = pltpu.async_copy(tableT_hbm.at[col], row_v, semr)
            row_copy.wait()
            pending = [None, None]
            for k in range(n_chunks):
                b = k % 2
                if pending[b] is not None:
                    pending[b].wait()

                @plsc.parallel_loop(0, _CHUNK // _LANES, unroll=8)
                def per_vec(j, k=k, b=b):
                    sl_out = pl.ds(j * _LANES, _LANES)
                    sl_idx = pl.ds(k * _CHUNK + j * _LANES, _LANES)
                    outc_v[b, sl_out] = plsc.load_gather(row_v, [idx_v[sl_idx]])
                pending[b] = pltpu.async_copy(
                    outc_v.at[b],
                    outT_hbm.at[col, pl.ds(k * _CHUNK, _CHUNK)],
                    sems[b],
                )
            for p in pending:
                if p is not None:
                    p.wait()

    return gather_kernel


@jax.jit
def kernel(x, pe):
    rows, dim = pe.shape
    gather = _build_gather(x.shape[0], dim, rows, pe.dtype.name)
    outT = gather(pe.T, x)
    return outT.T


# speedup vs baseline: 1.0595x; 1.0275x over previous
import functools

import jax
import jax.numpy as jnp
from jax import lax
from jax.experimental import pallas as pl
from jax.experimental.pallas import tpu as pltpu
from jax.experimental.pallas import tpu_sc as plsc

_NUM_CORES = 2
_NUM_SUBCORES = 16
_LANES = 16
_CHUNK = 4096


@functools.lru_cache(maxsize=None)
def _build_gather(batch, dim, rows, dtype_name):
    dtype = jnp.dtype(dtype_name)
    n_workers = _NUM_CORES * _NUM_SUBCORES
    cols_per_w = dim // n_workers
    n_chunks = batch // _CHUNK
    mesh = plsc.VectorSubcoreMesh(
        core_axis_name="c",
        subcore_axis_name="s",
        num_cores=_NUM_CORES,
        num_subcores=_NUM_SUBCORES,
    )

    @functools.partial(
        pl.kernel,
        mesh=mesh,
        out_type=jax.ShapeDtypeStruct((dim, batch), dtype),
        scratch_types=[
            pltpu.VMEM((rows,), dtype),
            pltpu.VMEM((batch,), jnp.int32),
            pltpu.VMEM((2, _CHUNK), dtype),
            pltpu.SemaphoreType.DMA,
            pltpu.SemaphoreType.DMA,
            pltpu.SemaphoreType.DMA,
        ],
        compiler_params=pltpu.CompilerParams(
            use_tc_tiling_on_sc=True, needs_layout_passes=False
        ),
    )
    def gather_kernel(
        tableT_hbm, idx_hbm, outT_hbm, row_v, idx_v, outc_v, sem0, sem1, semr
    ):
        wid = lax.axis_index("s") * _NUM_CORES + lax.axis_index("c")
        sems = (sem0, sem1)
        idx_copy = pltpu.async_copy(idx_hbm, idx_v, sem0)
        row_copy = pltpu.async_copy(tableT_hbm.at[wid * cols_per_w], row_v, semr)
        idx_copy.wait()

        for r in range(cols_per_w):
            col = wid * cols_per_w + r
            if r > 0:
                row_copy = pltpu.async_copy(tableT_hbm.at[col], row_v, semr)
            row_copy.wait()
            pending = [None, None]
            for k in range(n_chunks):
                b = k % 2
                if pending[b] is not None:
                    pending[b].wait()

                @plsc.parallel_loop(0, _CHUNK // _LANES, unroll=8)
                def per_vec(j, k=k, b=b):
                    sl_out = pl.ds(j * _LANES, _LANES)
                    sl_idx = pl.ds(k * _CHUNK + j * _LANES, _LANES)
                    outc_v[b, sl_out] = plsc.load_gather(row_v, [idx_v[sl_idx]])
                pending[b] = pltpu.async_copy(
                    outc_v.at[b],
                    outT_hbm.at[col, pl.ds(k * _CHUNK, _CHUNK)],
                    sems[b],
                )
            for p in pending:
                if p is not None:
                    p.wait()

    return gather_kernel


@jax.jit
def kernel(x, pe):
    rows, dim = pe.shape
    gather = _build_gather(x.shape[0], dim, rows, pe.dtype.name)
    outT = gather(pe.T, x)
    return outT.T
